# baseline (speedup 1.0000x reference)
import functools

import jax
import jax.numpy as jnp
import numpy as np
from jax import lax
from jax.experimental import pallas as pl
from jax.experimental.pallas import tpu as pltpu
from jax.experimental.pallas import tpu_sc as plsc

N = 10000
E = 320000
DF = 128
DE = 16
H = 8
HD = 16
SCALE = HD ** (-0.5)

NC = 2
NS = 16
NT = NC * NS
C = 32
G = C // 16
SUP = 12
NPAIR = SUP // 2
NSUP = 26
BCH = NSUP * SUP
NP_ = 10112
RS = NP_ // NS


def _proj_body(q_ref, k_ref, v_ref, wq_ref, wk_ref, wv_ref,
               qh_ref, kh_ref, vh_ref):
    qh_ref[...] = jnp.dot(q_ref[...], wq_ref[...],
                          preferred_element_type=jnp.float32) * SCALE
    kh_ref[...] = jnp.dot(k_ref[...], wk_ref[...],
                          preferred_element_type=jnp.float32)
    vh_ref[...] = jnp.dot(v_ref[...], wv_ref[...],
                          preferred_element_type=jnp.float32)


def _proj(q, k, v, Wq, Wk, Wv):
    BN = 2000
    grid = (N // BN,)
    bspec_x = pl.BlockSpec((BN, DF), lambda i: (i, 0))
    bspec_w = pl.BlockSpec((DF, DF), lambda i: (0, 0))
    return pl.pallas_call(
        _proj_body,
        grid=grid,
        in_specs=[bspec_x, bspec_x, bspec_x, bspec_w, bspec_w, bspec_w],
        out_specs=[bspec_x, bspec_x, bspec_x],
        out_shape=[jax.ShapeDtypeStruct((N, DF), jnp.float32)] * 3,
    )(q, k, v, Wq, Wk, Wv)


def _bias_body(e_ref, wb_ref, bb_ref, o_ref):
    o_ref[...] = jnp.dot(e_ref[...], wb_ref[...],
                         preferred_element_type=jnp.float32) + bb_ref[...]


def _bias(edges, Wb, bb):
    BE = 20000
    grid = (E // BE,)
    return pl.pallas_call(
        _bias_body,
        grid=grid,
        in_specs=[pl.BlockSpec((BE, DE), lambda i: (i, 0)),
                  pl.BlockSpec((DE, H), lambda i: (0, 0)),
                  pl.BlockSpec((1, H), lambda i: (0, 0))],
        out_specs=pl.BlockSpec((BE, H), lambda i: (i, 0)),
        out_shape=jax.ShapeDtypeStruct((E, H), jnp.float32),
    )(edges, Wb, bb.reshape(1, H))


def _sc_body(qh_hbm, kh_hbm, vh_hbm, bias_hbm, src2_hbm, dst2_hbm,
             acc_out, den_out,
             src2, dst2, bias2, qA, kA, vA, qB, kB, vB, exbuf,
             acc_sh, den_sh,
             gq0, gk0, gv0, gq1, gk1, gv1, sac0, sde0, sac1, sde1):
    c = lax.axis_index("c")
    s = lax.axis_index("s")
    tile = c * NS + s
    base_chunk = tile * BCH + jnp.minimum(tile, 16)

    iota = lax.iota(jnp.int32, 16)
    zero16 = jnp.zeros((16,), jnp.float32)

    def _zero_vrow(r, _):
        for j in range(DF // 16):
            vA[r, pl.ds(j * 16, 16)] = zero16
        exbuf[r, pl.ds(0, 16)] = zero16
        return 0

    lax.fori_loop(0, C, _zero_vrow, 0)

    row0 = s * RS
    for b in range(RS // C):
        pltpu.sync_copy(vA, acc_sh.at[pl.ds(row0 + b * C, C)])
        pltpu.sync_copy(exbuf, den_sh.at[pl.ds(row0 + b * C, C)])
    rtail = RS % C
    pltpu.sync_copy(vA.at[pl.ds(0, rtail)],
                    acc_sh.at[pl.ds(row0 + RS - rtail, rtail)])
    pltpu.sync_copy(exbuf.at[pl.ds(0, rtail)],
                    den_sh.at[pl.ds(row0 + RS - rtail, rtail)])
    plsc.subcore_barrier()

    def _issue(r, qb, kb, vb, sq, sk, sv):
        pltpu.async_copy(qh_hbm.at[src2.at[r]], qb, sq)
        pltpu.async_copy(kh_hbm.at[dst2.at[r]], kb, sk)
        pltpu.async_copy(vh_hbm.at[dst2.at[r]], vb, sv)

    def _wait_g(qb, kb, vb, sq, sk, sv):
        pltpu.make_async_copy(qh_hbm.at[src2.at[0]], qb, sq).wait()
        pltpu.make_async_copy(kh_hbm.at[dst2.at[0]], kb, sk).wait()
        pltpu.make_async_copy(vh_hbm.at[dst2.at[0]], vb, sv).wait()

    def _compute(r, qb, kb, vb):
        boff = r * (C * H)

        @plsc.parallel_loop(0, G * H, unroll=2)
        def _gh(i):
            g = i >> 3
            h = i & 7
            rows = g * 16 + iota
            hb = h * HD
            att0 = plsc.load_gather(bias2, [boff + rows * H + h])
            att1 = jnp.zeros((16,), jnp.float32)
            for d in range(0, HD, 2):
                c0 = jnp.full((16,), d, jnp.int32) + hb
                c1 = jnp.full((16,), d + 1, jnp.int32) + hb
                att0 = att0 + (plsc.load_gather(qb, [rows, c0])
                               * plsc.load_gather(kb, [rows, c0]))
                att1 = att1 + (plsc.load_gather(qb, [rows, c1])
                               * plsc.load_gather(kb, [rows, c1]))
            ex = jnp.exp(att0 + att1)
            plsc.store_scatter(exbuf, [rows, jnp.full((16,), 0, jnp.int32) + h], ex)
            for d in range(HD):
                col = jnp.full((16,), d, jnp.int32) + hb
                vc = plsc.load_gather(vb, [rows, col])
                plsc.store_scatter(vb, [rows, col], vc * ex)

    def _scat(r, vb, sa, sd):
        pltpu.async_copy(vb, acc_sh.at[src2.at[r]], sa, add=True)
        pltpu.async_copy(exbuf, den_sh.at[src2.at[r]], sd, add=True)

    def _wait_scat(vb, sa, sd):
        pltpu.make_async_copy(vb, acc_sh.at[src2.at[0]], sa).wait()
        pltpu.make_async_copy(exbuf, den_sh.at[src2.at[0]], sd).wait()

    def _sup(k, _):
        off = base_chunk + k * SUP
        pltpu.sync_copy(src2_hbm.at[pl.ds(off, SUP)], src2)
        pltpu.sync_copy(dst2_hbm.at[pl.ds(off, SUP)], dst2)
        pltpu.sync_copy(bias_hbm.at[pl.ds(off * (C * H), SUP * C * H)], bias2)
        _issue(0, qA, kA, vA, gq0, gk0, gv0)

        def _pair(j, _):
            a = 2 * j
            _wait_g(qA, kA, vA, gq0, gk0, gv0)
            _issue(a + 1, qB, kB, vB, gq1, gk1, gv1)
            _compute(a, qA, kA, vA)
            _scat(a, vA, sac0, sde0)
            _wait_g(qB, kB, vB, gq1, gk1, gv1)
            _wait_scat(vA, sac0, sde0)
            _compute(a + 1, qB, kB, vB)
            _scat(a + 1, vB, sac1, sde1)

            @pl.when(j < NPAIR - 1)
            def _():
                _issue(a + 2, qA, kA, vA, gq0, gk0, gv0)

            _wait_scat(vB, sac1, sde1)
            return 0

        lax.fori_loop(0, NPAIR, _pair, 0)
        return 0

    lax.fori_loop(0, NSUP, _sup, 0)

    @pl.when(tile < 16)
    def _():
        off = base_chunk + BCH
        pltpu.sync_copy(src2_hbm.at[pl.ds(off, 1)], src2.at[pl.ds(0, 1)])
        pltpu.sync_copy(dst2_hbm.at[pl.ds(off, 1)], dst2.at[pl.ds(0, 1)])
        pltpu.sync_copy(bias_hbm.at[pl.ds(off * (C * H), C * H)],
                        bias2.at[pl.ds(0, C * H)])
        _issue(0, qA, kA, vA, gq0, gk0, gv0)
        _wait_g(qA, kA, vA, gq0, gk0, gv0)
        _compute(0, qA, kA, vA)
        _scat(0, vA, sac0, sde0)
        _wait_scat(vA, sac0, sde0)

    plsc.subcore_barrier()

    pltpu.sync_copy(acc_sh.at[pl.ds(row0, RS)], acc_out.at[c, pl.ds(row0, RS)])
    pltpu.sync_copy(den_sh.at[pl.ds(row0, RS)], den_out.at[c, pl.ds(row0, RS)])


def _sc_pass(qh, kh, vh, bias1d, src, dst):
    mesh = plsc.VectorSubcoreMesh(core_axis_name="c", subcore_axis_name="s")
    f = pl.kernel(
        _sc_body,
        out_type=(jax.ShapeDtypeStruct((NC, NP_, DF), jnp.float32),
                  jax.ShapeDtypeStruct((NC, NP_, 2 * H), jnp.float32)),
        mesh=mesh,
        compiler_params=pltpu.CompilerParams(needs_layout_passes=False,
                                             use_tc_tiling_on_sc=False),
        scratch_types=[
            pltpu.VMEM((SUP, C), jnp.int32),
            pltpu.VMEM((SUP, C), jnp.int32),
            pltpu.VMEM((SUP * C * H,), jnp.float32),
            pltpu.VMEM((C, DF), jnp.float32),
            pltpu.VMEM((C, DF), jnp.float32),
            pltpu.VMEM((C, DF), jnp.float32),
            pltpu.VMEM((C, DF), jnp.float32),
            pltpu.VMEM((C, DF), jnp.float32),
            pltpu.VMEM((C, DF), jnp.float32),
            pltpu.VMEM((C, 2 * H), jnp.float32),
            pltpu.VMEM_SHARED((NP_, DF), jnp.float32),
            pltpu.VMEM_SHARED((NP_, 2 * H), jnp.float32),
        ] + [pltpu.SemaphoreType.DMA] * 10,
    )
    return f(qh, kh, vh, bias1d,
             src.reshape(E // C, C), dst.reshape(E // C, C))


def _final_body(acc_ref, den_ref, r_ref, wo_ref, bo_ref, o_ref):
    a = acc_ref[0] + acc_ref[1]
    dn = den_ref[0] + den_ref[1]
    dr = jnp.dot(dn, r_ref[...], preferred_element_type=jnp.float32)
    dr = jnp.where(dr == 0.0, 1.0, dr)
    o = a / dr
    o_ref[...] = jnp.dot(o, wo_ref[...],
                         preferred_element_type=jnp.float32) + bo_ref[...]


def _finalize(acc, den, Wo, bo):
    BN = 2000
    grid = (N // BN,)
    rep = jnp.asarray(
        np.vstack([np.kron(np.eye(H), np.ones((1, HD))),
                   np.zeros((H, DF))]), dtype=jnp.float32)
    return pl.pallas_call(
        _final_body,
        grid=grid,
        in_specs=[pl.BlockSpec((NC, BN, DF), lambda i: (0, i, 0)),
                  pl.BlockSpec((NC, BN, 2 * H), lambda i: (0, i, 0)),
                  pl.BlockSpec((2 * H, DF), lambda i: (0, 0)),
                  pl.BlockSpec((DF, DF), lambda i: (0, 0)),
                  pl.BlockSpec((1, DF), lambda i: (0, 0))],
        out_specs=pl.BlockSpec((BN, DF), lambda i: (i, 0)),
        out_shape=jax.ShapeDtypeStruct((N, DF), jnp.float32),
    )(acc, den, rep, Wo, bo.reshape(1, DF))


def kernel(q, k, v, edges, edge_index, Wq, Wk, Wv, Wo, bo, Wb, bb):
    src = edge_index[:, 0]
    dst = edge_index[:, 1]
    qh, kh, vh = _proj(q, k, v, Wq, Wk, Wv)
    bias = _bias(edges, Wb, bb)
    acc, den = _sc_pass(qh, kh, vh, bias.reshape(E * H), src, dst)
    return _finalize(acc, den, Wo, bo)


# speedup vs baseline: 1.0907x; 1.0907x over previous
import functools

import jax
import jax.numpy as jnp
import numpy as np
from jax import lax
from jax.experimental import pallas as pl
from jax.experimental.pallas import tpu as pltpu
from jax.experimental.pallas import tpu_sc as plsc

N = 10000
E = 320000
DF = 128
DE = 16
H = 8
HD = 16
SCALE = HD ** (-0.5)

NC = 2
NS = 16
NT = NC * NS
C = 32
G = C // 16
SUP = 12
NPAIR = SUP // 2
NSUP = 26
BCH = NSUP * SUP
NP_ = 10112
RS = NP_ // NS


def _proj_body(q_ref, k_ref, v_ref, wq_ref, wk_ref, wv_ref,
               qh_ref, kh_ref, vh_ref):
    qh_ref[...] = jnp.dot(q_ref[...], wq_ref[...],
                          preferred_element_type=jnp.float32) * SCALE
    kh_ref[...] = jnp.dot(k_ref[...], wk_ref[...],
                          preferred_element_type=jnp.float32)
    vh_ref[...] = jnp.dot(v_ref[...], wv_ref[...],
                          preferred_element_type=jnp.float32)


def _proj(q, k, v, Wq, Wk, Wv):
    BN = 2000
    grid = (N // BN,)
    bspec_x = pl.BlockSpec((BN, DF), lambda i: (i, 0))
    bspec_w = pl.BlockSpec((DF, DF), lambda i: (0, 0))
    return pl.pallas_call(
        _proj_body,
        grid=grid,
        in_specs=[bspec_x, bspec_x, bspec_x, bspec_w, bspec_w, bspec_w],
        out_specs=[bspec_x, bspec_x, bspec_x],
        out_shape=[jax.ShapeDtypeStruct((N, DF), jnp.float32)] * 3,
    )(q, k, v, Wq, Wk, Wv)


def _bias_body(e_ref, wb_ref, bb_ref, o_ref):
    o_ref[...] = jnp.dot(e_ref[...], wb_ref[...],
                         preferred_element_type=jnp.float32) + bb_ref[...]


def _bias(edges, Wb, bb):
    BE = 20000
    grid = (E // BE,)
    return pl.pallas_call(
        _bias_body,
        grid=grid,
        in_specs=[pl.BlockSpec((BE, DE), lambda i: (i, 0)),
                  pl.BlockSpec((DE, H), lambda i: (0, 0)),
                  pl.BlockSpec((1, H), lambda i: (0, 0))],
        out_specs=pl.BlockSpec((BE, H), lambda i: (i, 0)),
        out_shape=jax.ShapeDtypeStruct((E, H), jnp.float32),
    )(edges, Wb, bb.reshape(1, H))


def _sc_body(qh_hbm, kh_hbm, vh_hbm, bias_hbm, src2_hbm, dst2_hbm,
             acc_out, den_out,
             src2, dst2, bias2, qA, kA, vA, qB, kB, vB, exbuf,
             acc_sh, den_sh,
             gq0, gk0, gv0, gq1, gk1, gv1, sac0, sde0, sac1, sde1):
    c = lax.axis_index("c")
    s = lax.axis_index("s")
    tile = c * NS + s
    base_chunk = tile * BCH + jnp.minimum(tile, 16)

    iota = lax.iota(jnp.int32, 16)
    zero16 = jnp.zeros((16,), jnp.float32)

    def _zero_vrow(r, _):
        for j in range(DF // 16):
            vA[r, pl.ds(j * 16, 16)] = zero16
        exbuf[r, pl.ds(0, 16)] = zero16
        return 0

    lax.fori_loop(0, C, _zero_vrow, 0)

    row0 = s * RS
    for b in range(RS // C):
        pltpu.sync_copy(vA, acc_sh.at[pl.ds(row0 + b * C, C)])
        pltpu.sync_copy(exbuf, den_sh.at[pl.ds(row0 + b * C, C)])
    rtail = RS % C
    pltpu.sync_copy(vA.at[pl.ds(0, rtail)],
                    acc_sh.at[pl.ds(row0 + RS - rtail, rtail)])
    pltpu.sync_copy(exbuf.at[pl.ds(0, rtail)],
                    den_sh.at[pl.ds(row0 + RS - rtail, rtail)])
    plsc.subcore_barrier()

    def _issue(r, qb, kb, vb, sq, sk, sv):
        pltpu.async_copy(qh_hbm.at[src2.at[r]], qb, sq)
        pltpu.async_copy(kh_hbm.at[dst2.at[r]], kb, sk)
        pltpu.async_copy(vh_hbm.at[dst2.at[r]], vb, sv)

    def _wait_g(qb, kb, vb, sq, sk, sv):
        pltpu.make_async_copy(qh_hbm.at[src2.at[0]], qb, sq).wait()
        pltpu.make_async_copy(kh_hbm.at[dst2.at[0]], kb, sk).wait()
        pltpu.make_async_copy(vh_hbm.at[dst2.at[0]], vb, sv).wait()

    def _compute(r, qb, kb, vb):
        boff = r * (C * H)

        @plsc.parallel_loop(0, G * H, unroll=2)
        def _dots(i):
            g = i >> 3
            h = i & 7
            rows = g * 16 + iota
            hb = h * HD
            att0 = plsc.load_gather(bias2, [boff + rows * H + h])
            att1 = jnp.zeros((16,), jnp.float32)
            for d in range(0, HD, 2):
                c0 = jnp.full((16,), d, jnp.int32) + hb
                c1 = jnp.full((16,), d + 1, jnp.int32) + hb
                att0 = att0 + (plsc.load_gather(qb, [rows, c0])
                               * plsc.load_gather(kb, [rows, c0]))
                att1 = att1 + (plsc.load_gather(qb, [rows, c1])
                               * plsc.load_gather(kb, [rows, c1]))
            ex = jnp.exp(att0 + att1)
            plsc.store_scatter(exbuf, [rows, jnp.full((16,), 0, jnp.int32) + h], ex)

        @plsc.parallel_loop(0, G * H * 2, unroll=4)
        def _scale(i):
            g = i >> 4
            h = (i >> 1) & 7
            rows = g * 16 + iota
            hb = h * HD + (i & 1) * (HD // 2)
            ex = plsc.load_gather(exbuf, [rows, jnp.full((16,), 0, jnp.int32) + h])
            for d in range(HD // 2):
                col = jnp.full((16,), d, jnp.int32) + hb
                vc = plsc.load_gather(vb, [rows, col])
                plsc.store_scatter(vb, [rows, col], vc * ex)

    def _scat(r, vb, sa, sd):
        pltpu.async_copy(vb, acc_sh.at[src2.at[r]], sa, add=True)
        pltpu.async_copy(exbuf, den_sh.at[src2.at[r]], sd, add=True)

    def _wait_scat(vb, sa, sd):
        pltpu.make_async_copy(vb, acc_sh.at[src2.at[0]], sa).wait()
        pltpu.make_async_copy(exbuf, den_sh.at[src2.at[0]], sd).wait()

    def _sup(k, _):
        off = base_chunk + k * SUP
        pltpu.sync_copy(src2_hbm.at[pl.ds(off, SUP)], src2)
        pltpu.sync_copy(dst2_hbm.at[pl.ds(off, SUP)], dst2)
        pltpu.sync_copy(bias_hbm.at[pl.ds(off * (C * H), SUP * C * H)], bias2)
        _issue(0, qA, kA, vA, gq0, gk0, gv0)

        def _pair(j, _):
            a = 2 * j
            _wait_g(qA, kA, vA, gq0, gk0, gv0)
            _issue(a + 1, qB, kB, vB, gq1, gk1, gv1)
            _compute(a, qA, kA, vA)
            _scat(a, vA, sac0, sde0)
            _wait_g(qB, kB, vB, gq1, gk1, gv1)
            _wait_scat(vA, sac0, sde0)
            _compute(a + 1, qB, kB, vB)
            _scat(a + 1, vB, sac1, sde1)

            @pl.when(j < NPAIR - 1)
            def _():
                _issue(a + 2, qA, kA, vA, gq0, gk0, gv0)

            _wait_scat(vB, sac1, sde1)
            return 0

        lax.fori_loop(0, NPAIR, _pair, 0)
        return 0

    lax.fori_loop(0, NSUP, _sup, 0)

    @pl.when(tile < 16)
    def _():
        off = base_chunk + BCH
        pltpu.sync_copy(src2_hbm.at[pl.ds(off, 1)], src2.at[pl.ds(0, 1)])
        pltpu.sync_copy(dst2_hbm.at[pl.ds(off, 1)], dst2.at[pl.ds(0, 1)])
        pltpu.sync_copy(bias_hbm.at[pl.ds(off * (C * H), C * H)],
                        bias2.at[pl.ds(0, C * H)])
        _issue(0, qA, kA, vA, gq0, gk0, gv0)
        _wait_g(qA, kA, vA, gq0, gk0, gv0)
        _compute(0, qA, kA, vA)
        _scat(0, vA, sac0, sde0)
        _wait_scat(vA, sac0, sde0)

    plsc.subcore_barrier()

    pltpu.sync_copy(acc_sh.at[pl.ds(row0, RS)], acc_out.at[c, pl.ds(row0, RS)])
    pltpu.sync_copy(den_sh.at[pl.ds(row0, RS)], den_out.at[c, pl.ds(row0, RS)])


def _sc_pass(qh, kh, vh, bias1d, src, dst):
    mesh = plsc.VectorSubcoreMesh(core_axis_name="c", subcore_axis_name="s")
    f = pl.kernel(
        _sc_body,
        out_type=(jax.ShapeDtypeStruct((NC, NP_, DF), jnp.float32),
                  jax.ShapeDtypeStruct((NC, NP_, 2 * H), jnp.float32)),
        mesh=mesh,
        compiler_params=pltpu.CompilerParams(needs_layout_passes=False,
                                             use_tc_tiling_on_sc=False),
        scratch_types=[
            pltpu.VMEM((SUP, C), jnp.int32),
            pltpu.VMEM((SUP, C), jnp.int32),
            pltpu.VMEM((SUP * C * H,), jnp.float32),
            pltpu.VMEM((C, DF), jnp.float32),
            pltpu.VMEM((C, DF), jnp.float32),
            pltpu.VMEM((C, DF), jnp.float32),
            pltpu.VMEM((C, DF), jnp.float32),
            pltpu.VMEM((C, DF), jnp.float32),
            pltpu.VMEM((C, DF), jnp.float32),
            pltpu.VMEM((C, 2 * H), jnp.float32),
            pltpu.VMEM_SHARED((NP_, DF), jnp.float32),
            pltpu.VMEM_SHARED((NP_, 2 * H), jnp.float32),
        ] + [pltpu.SemaphoreType.DMA] * 10,
    )
    return f(qh, kh, vh, bias1d,
             src.reshape(E // C, C), dst.reshape(E // C, C))


def _final_body(acc_ref, den_ref, r_ref, wo_ref, bo_ref, o_ref):
    a = acc_ref[0] + acc_ref[1]
    dn = den_ref[0] + den_ref[1]
    dr = jnp.dot(dn, r_ref[...], preferred_element_type=jnp.float32)
    dr = jnp.where(dr == 0.0, 1.0, dr)
    o = a / dr
    o_ref[...] = jnp.dot(o, wo_ref[...],
                         preferred_element_type=jnp.float32) + bo_ref[...]


def _finalize(acc, den, Wo, bo):
    BN = 2000
    grid = (N // BN,)
    rep = jnp.asarray(
        np.vstack([np.kron(np.eye(H), np.ones((1, HD))),
                   np.zeros((H, DF))]), dtype=jnp.float32)
    return pl.pallas_call(
        _final_body,
        grid=grid,
        in_specs=[pl.BlockSpec((NC, BN, DF), lambda i: (0, i, 0)),
                  pl.BlockSpec((NC, BN, 2 * H), lambda i: (0, i, 0)),
                  pl.BlockSpec((2 * H, DF), lambda i: (0, 0)),
                  pl.BlockSpec((DF, DF), lambda i: (0, 0)),
                  pl.BlockSpec((1, DF), lambda i: (0, 0))],
        out_specs=pl.BlockSpec((BN, DF), lambda i: (i, 0)),
        out_shape=jax.ShapeDtypeStruct((N, DF), jnp.float32),
    )(acc, den, rep, Wo, bo.reshape(1, DF))


def kernel(q, k, v, edges, edge_index, Wq, Wk, Wv, Wo, bo, Wb, bb):
    src = edge_index[:, 0]
    dst = edge_index[:, 1]
    qh, kh, vh = _proj(q, k, v, Wq, Wk, Wv)
    bias = _bias(edges, Wb, bb)
    acc, den = _sc_pass(qh, kh, vh, bias.reshape(E * H), src, dst)
    return _finalize(acc, den, Wo, bo)


# speedup vs baseline: 3.2488x; 2.9786x over previous
import functools

import jax
import jax.numpy as jnp
import numpy as np
from jax import lax
from jax.experimental import pallas as pl
from jax.experimental.pallas import tpu as pltpu
from jax.experimental.pallas import tpu_sc as plsc

N = 10000
E = 320000
DF = 128
DE = 16
H = 8
HD = 16
SCALE = HD ** (-0.5)

NC = 2
NS = 16
NT = NC * NS
C = 32
G = C // 16
SUP = 12
NPAIR = SUP // 2
NSUP = 26
BCH = NSUP * SUP
NP_ = 10112
RS = NP_ // NS


def _proj_body(q_ref, k_ref, v_ref, wq_ref, wk_ref, wv_ref,
               qh_ref, kh_ref, vh_ref):
    qh_ref[...] = jnp.dot(q_ref[...], wq_ref[...],
                          preferred_element_type=jnp.float32) * SCALE
    kh_ref[...] = jnp.dot(k_ref[...], wk_ref[...],
                          preferred_element_type=jnp.float32)
    vh_ref[...] = jnp.dot(v_ref[...], wv_ref[...],
                          preferred_element_type=jnp.float32)


def _proj(q, k, v, Wq, Wk, Wv):
    BN = 2000
    grid = (N // BN,)
    bspec_x = pl.BlockSpec((BN, DF), lambda i: (i, 0))
    bspec_w = pl.BlockSpec((DF, DF), lambda i: (0, 0))
    return pl.pallas_call(
        _proj_body,
        grid=grid,
        in_specs=[bspec_x, bspec_x, bspec_x, bspec_w, bspec_w, bspec_w],
        out_specs=[bspec_x, bspec_x, bspec_x],
        out_shape=[jax.ShapeDtypeStruct((N, DF), jnp.float32)] * 3,
    )(q, k, v, Wq, Wk, Wv)


def _bias_body(e_ref, wb_ref, bb_ref, o_ref):
    o_ref[...] = jnp.dot(e_ref[...], wb_ref[...],
                         preferred_element_type=jnp.float32) + bb_ref[...]


def _bias(edges, Wb, bb):
    BE = 20000
    grid = (E // BE,)
    return pl.pallas_call(
        _bias_body,
        grid=grid,
        in_specs=[pl.BlockSpec((BE, DE), lambda i: (i, 0)),
                  pl.BlockSpec((DE, H), lambda i: (0, 0)),
                  pl.BlockSpec((1, H), lambda i: (0, 0))],
        out_specs=pl.BlockSpec((BE, H), lambda i: (i, 0)),
        out_shape=jax.ShapeDtypeStruct((E, H), jnp.float32),
    )(edges, Wb, bb.reshape(1, H))


def _sc_body(qh_hbm, kh_hbm, vh_hbm, bias_hbm, src2_hbm, dst2_hbm,
             acc_out, den_out,
             src2, dst2, bias2, qA, kA, vA, qB, kB, vB, exbuf,
             acc_sh, den_sh,
             gq0, gk0, gv0, gq1, gk1, gv1, sac0, sde0, sac1, sde1):
    c = lax.axis_index("c")
    s = lax.axis_index("s")
    tile = c * NS + s
    base_chunk = tile * BCH + jnp.minimum(tile, 16)

    iota = lax.iota(jnp.int32, 16)
    zero16 = jnp.zeros((16,), jnp.float32)

    def _zero_vrow(r, _):
        for j in range(DF // 16):
            vA[r, pl.ds(j * 16, 16)] = zero16
        exbuf[r, pl.ds(0, 16)] = zero16
        return 0

    lax.fori_loop(0, C, _zero_vrow, 0)

    row0 = s * RS
    for b in range(RS // C):
        pltpu.sync_copy(vA, acc_sh.at[pl.ds(row0 + b * C, C)])
        pltpu.sync_copy(exbuf, den_sh.at[pl.ds(row0 + b * C, C)])
    rtail = RS % C
    pltpu.sync_copy(vA.at[pl.ds(0, rtail)],
                    acc_sh.at[pl.ds(row0 + RS - rtail, rtail)])
    pltpu.sync_copy(exbuf.at[pl.ds(0, rtail)],
                    den_sh.at[pl.ds(row0 + RS - rtail, rtail)])
    plsc.subcore_barrier()

    def _issue(r, qb, kb, vb, sq, sk, sv):
        pltpu.async_copy(qh_hbm.at[src2.at[r]], qb, sq)
        pltpu.async_copy(kh_hbm.at[dst2.at[r]], kb, sk)
        pltpu.async_copy(vh_hbm.at[dst2.at[r]], vb, sv)

    def _wait_g(qb, kb, vb, sq, sk, sv):
        pltpu.make_async_copy(qh_hbm.at[src2.at[0]], qb, sq).wait()
        pltpu.make_async_copy(kh_hbm.at[dst2.at[0]], kb, sk).wait()
        pltpu.make_async_copy(vh_hbm.at[dst2.at[0]], vb, sv).wait()

    def _compute(r, qb, kb, vb):
        boff = r * (C * H)

        @plsc.parallel_loop(0, G * H, unroll=2)
        def _dots(i):
            g = i >> 3
            h = i & 7
            rows = g * 16 + iota
            hb = h * HD
            att0 = plsc.load_gather(bias2, [boff + rows * H + h])
            att1 = jnp.zeros((16,), jnp.float32)
            for d in range(0, HD, 2):
                c0 = ((d + iota) & (HD - 1)) + hb
                c1 = ((d + 1 + iota) & (HD - 1)) + hb
                att0 = att0 + (plsc.load_gather(qb, [rows, c0])
                               * plsc.load_gather(kb, [rows, c0]))
                att1 = att1 + (plsc.load_gather(qb, [rows, c1])
                               * plsc.load_gather(kb, [rows, c1]))
            ex = jnp.exp(att0 + att1)
            plsc.store_scatter(exbuf, [rows, jnp.full((16,), 0, jnp.int32) + h], ex)

        @plsc.parallel_loop(0, G * H, unroll=2)
        def _scale(i):
            g = i >> 3
            h = i & 7
            rows = g * 16 + iota
            hb = h * HD
            ex = plsc.load_gather(exbuf, [rows, jnp.full((16,), 0, jnp.int32) + h])
            for d in range(HD):
                col = ((d + iota) & (HD - 1)) + hb
                vc = plsc.load_gather(vb, [rows, col])
                plsc.store_scatter(vb, [rows, col], vc * ex)

    def _scat(r, vb, sa, sd):
        pltpu.async_copy(vb, acc_sh.at[src2.at[r]], sa, add=True)
        pltpu.async_copy(exbuf, den_sh.at[src2.at[r]], sd, add=True)

    def _wait_scat(vb, sa, sd):
        pltpu.make_async_copy(vb, acc_sh.at[src2.at[0]], sa).wait()
        pltpu.make_async_copy(exbuf, den_sh.at[src2.at[0]], sd).wait()

    def _sup(k, _):
        off = base_chunk + k * SUP
        pltpu.sync_copy(src2_hbm.at[pl.ds(off, SUP)], src2)
        pltpu.sync_copy(dst2_hbm.at[pl.ds(off, SUP)], dst2)
        pltpu.sync_copy(bias_hbm.at[pl.ds(off * (C * H), SUP * C * H)], bias2)
        _issue(0, qA, kA, vA, gq0, gk0, gv0)

        def _pair(j, _):
            a = 2 * j
            _wait_g(qA, kA, vA, gq0, gk0, gv0)
            _issue(a + 1, qB, kB, vB, gq1, gk1, gv1)
            _compute(a, qA, kA, vA)
            _scat(a, vA, sac0, sde0)
            _wait_g(qB, kB, vB, gq1, gk1, gv1)
            _wait_scat(vA, sac0, sde0)
            _compute(a + 1, qB, kB, vB)
            _scat(a + 1, vB, sac1, sde1)

            @pl.when(j < NPAIR - 1)
            def _():
                _issue(a + 2, qA, kA, vA, gq0, gk0, gv0)

            _wait_scat(vB, sac1, sde1)
            return 0

        lax.fori_loop(0, NPAIR, _pair, 0)
        return 0

    lax.fori_loop(0, NSUP, _sup, 0)

    @pl.when(tile < 16)
    def _():
        off = base_chunk + BCH
        pltpu.sync_copy(src2_hbm.at[pl.ds(off, 1)], src2.at[pl.ds(0, 1)])
        pltpu.sync_copy(dst2_hbm.at[pl.ds(off, 1)], dst2.at[pl.ds(0, 1)])
        pltpu.sync_copy(bias_hbm.at[pl.ds(off * (C * H), C * H)],
                        bias2.at[pl.ds(0, C * H)])
        _issue(0, qA, kA, vA, gq0, gk0, gv0)
        _wait_g(qA, kA, vA, gq0, gk0, gv0)
        _compute(0, qA, kA, vA)
        _scat(0, vA, sac0, sde0)
        _wait_scat(vA, sac0, sde0)

    plsc.subcore_barrier()

    pltpu.sync_copy(acc_sh.at[pl.ds(row0, RS)], acc_out.at[c, pl.ds(row0, RS)])
    pltpu.sync_copy(den_sh.at[pl.ds(row0, RS)], den_out.at[c, pl.ds(row0, RS)])


def _sc_pass(qh, kh, vh, bias1d, src, dst):
    mesh = plsc.VectorSubcoreMesh(core_axis_name="c", subcore_axis_name="s")
    f = pl.kernel(
        _sc_body,
        out_type=(jax.ShapeDtypeStruct((NC, NP_, DF), jnp.float32),
                  jax.ShapeDtypeStruct((NC, NP_, 2 * H), jnp.float32)),
        mesh=mesh,
        compiler_params=pltpu.CompilerParams(needs_layout_passes=False,
                                             use_tc_tiling_on_sc=False),
        scratch_types=[
            pltpu.VMEM((SUP, C), jnp.int32),
            pltpu.VMEM((SUP, C), jnp.int32),
            pltpu.VMEM((SUP * C * H,), jnp.float32),
            pltpu.VMEM((C, DF), jnp.float32),
            pltpu.VMEM((C, DF), jnp.float32),
            pltpu.VMEM((C, DF), jnp.float32),
            pltpu.VMEM((C, DF), jnp.float32),
            pltpu.VMEM((C, DF), jnp.float32),
            pltpu.VMEM((C, DF), jnp.float32),
            pltpu.VMEM((C, 2 * H), jnp.float32),
            pltpu.VMEM_SHARED((NP_, DF), jnp.float32),
            pltpu.VMEM_SHARED((NP_, 2 * H), jnp.float32),
        ] + [pltpu.SemaphoreType.DMA] * 10,
    )
    return f(qh, kh, vh, bias1d,
             src.reshape(E // C, C), dst.reshape(E // C, C))


def _final_body(acc_ref, den_ref, r_ref, wo_ref, bo_ref, o_ref):
    a = acc_ref[0] + acc_ref[1]
    dn = den_ref[0] + den_ref[1]
    dr = jnp.dot(dn, r_ref[...], preferred_element_type=jnp.float32)
    dr = jnp.where(dr == 0.0, 1.0, dr)
    o = a / dr
    o_ref[...] = jnp.dot(o, wo_ref[...],
                         preferred_element_type=jnp.float32) + bo_ref[...]


def _finalize(acc, den, Wo, bo):
    BN = 2000
    grid = (N // BN,)
    rep = jnp.asarray(
        np.vstack([np.kron(np.eye(H), np.ones((1, HD))),
                   np.zeros((H, DF))]), dtype=jnp.float32)
    return pl.pallas_call(
        _final_body,
        grid=grid,
        in_specs=[pl.BlockSpec((NC, BN, DF), lambda i: (0, i, 0)),
                  pl.BlockSpec((NC, BN, 2 * H), lambda i: (0, i, 0)),
                  pl.BlockSpec((2 * H, DF), lambda i: (0, 0)),
                  pl.BlockSpec((DF, DF), lambda i: (0, 0)),
                  pl.BlockSpec((1, DF), lambda i: (0, 0))],
        out_specs=pl.BlockSpec((BN, DF), lambda i: (i, 0)),
        out_shape=jax.ShapeDtypeStruct((N, DF), jnp.float32),
    )(acc, den, rep, Wo, bo.reshape(1, DF))


def kernel(q, k, v, edges, edge_index, Wq, Wk, Wv, Wo, bo, Wb, bb):
    src = edge_index[:, 0]
    dst = edge_index[:, 1]
    qh, kh, vh = _proj(q, k, v, Wq, Wk, Wv)
    bias = _bias(edges, Wb, bb)
    acc, den = _sc_pass(qh, kh, vh, bias.reshape(E * H), src, dst)
    return _finalize(acc, den, Wo, bo)


# speedup vs baseline: 3.8017x; 1.1702x over previous
import functools

import jax
import jax.numpy as jnp
import numpy as np
from jax import lax
from jax.experimental import pallas as pl
from jax.experimental.pallas import tpu as pltpu
from jax.experimental.pallas import tpu_sc as plsc

N = 10000
E = 320000
DF = 128
DE = 16
H = 8
HD = 16
SCALE = HD ** (-0.5)

NC = 2
NS = 16
NT = NC * NS
C = 32
G = C // 16
SUP = 24
NPAIR = SUP // 2
NSUP = 13
BCH = NSUP * SUP
NP_ = 10112
RS = NP_ // NS


def _proj_body(q_ref, k_ref, v_ref, wq_ref, wk_ref, wv_ref,
               qh_ref, kh_ref, vh_ref):
    qh_ref[...] = jnp.dot(q_ref[...], wq_ref[...],
                          preferred_element_type=jnp.float32) * SCALE
    kh_ref[...] = jnp.dot(k_ref[...], wk_ref[...],
                          preferred_element_type=jnp.float32)
    vh_ref[...] = jnp.dot(v_ref[...], wv_ref[...],
                          preferred_element_type=jnp.float32)


def _proj(q, k, v, Wq, Wk, Wv):
    BN = 2000
    grid = (N // BN,)
    bspec_x = pl.BlockSpec((BN, DF), lambda i: (i, 0))
    bspec_w = pl.BlockSpec((DF, DF), lambda i: (0, 0))
    return pl.pallas_call(
        _proj_body,
        grid=grid,
        in_specs=[bspec_x, bspec_x, bspec_x, bspec_w, bspec_w, bspec_w],
        out_specs=[bspec_x, bspec_x, bspec_x],
        out_shape=[jax.ShapeDtypeStruct((N, DF), jnp.float32)] * 3,
    )(q, k, v, Wq, Wk, Wv)


def _bias_body(e_ref, wb_ref, bb_ref, o_ref):
    o_ref[...] = jnp.dot(e_ref[...], wb_ref[...],
                         preferred_element_type=jnp.float32) + bb_ref[...]


def _bias(edges, Wb, bb):
    BE = 20000
    grid = (E // BE,)
    return pl.pallas_call(
        _bias_body,
        grid=grid,
        in_specs=[pl.BlockSpec((BE, DE), lambda i: (i, 0)),
                  pl.BlockSpec((DE, H), lambda i: (0, 0)),
                  pl.BlockSpec((1, H), lambda i: (0, 0))],
        out_specs=pl.BlockSpec((BE, H), lambda i: (i, 0)),
        out_shape=jax.ShapeDtypeStruct((E, H), jnp.float32),
    )(edges, Wb, bb.reshape(1, H))


def _sc_body(qh_hbm, kh_hbm, vh_hbm, bias_hbm, src2_hbm, dst2_hbm,
             acc_out, den_out,
             src2, dst2, bias2, qA, kA, vA, qB, kB, vB, exA, exB,
             acc_sh, den_sh,
             gq0, gk0, gv0, gq1, gk1, gv1, sac0, sde0, sac1, sde1):
    c = lax.axis_index("c")
    s = lax.axis_index("s")
    tile = c * NS + s
    base_chunk = tile * BCH + jnp.minimum(tile, 16)

    iota = lax.iota(jnp.int32, 16)
    zero16 = jnp.zeros((16,), jnp.float32)

    def _zero_vrow(r, _):
        for j in range(DF // 16):
            vA[r, pl.ds(j * 16, 16)] = zero16
        exA[r, pl.ds(0, 16)] = zero16
        exB[r, pl.ds(0, 16)] = zero16
        return 0

    lax.fori_loop(0, C, _zero_vrow, 0)

    row0 = s * RS
    for b in range(RS // C):
        pltpu.sync_copy(vA, acc_sh.at[pl.ds(row0 + b * C, C)])
        pltpu.sync_copy(exA, den_sh.at[pl.ds(row0 + b * C, C)])
    rtail = RS % C
    pltpu.sync_copy(vA.at[pl.ds(0, rtail)],
                    acc_sh.at[pl.ds(row0 + RS - rtail, rtail)])
    pltpu.sync_copy(exA.at[pl.ds(0, rtail)],
                    den_sh.at[pl.ds(row0 + RS - rtail, rtail)])
    plsc.subcore_barrier()

    def _issue(r, qb, kb, vb, sq, sk, sv):
        pltpu.async_copy(qh_hbm.at[src2.at[r]], qb, sq)
        pltpu.async_copy(kh_hbm.at[dst2.at[r]], kb, sk)
        pltpu.async_copy(vh_hbm.at[dst2.at[r]], vb, sv)

    def _wait_g(qb, kb, vb, sq, sk, sv):
        pltpu.make_async_copy(qh_hbm.at[src2.at[0]], qb, sq).wait()
        pltpu.make_async_copy(kh_hbm.at[dst2.at[0]], kb, sk).wait()
        pltpu.make_async_copy(vh_hbm.at[dst2.at[0]], vb, sv).wait()

    def _compute(r, qb, kb, vb, exbuf):
        boff = r * (C * H)

        @plsc.parallel_loop(0, G * H, unroll=2)
        def _dots(i):
            g = i >> 3
            h = i & 7
            rows = g * 16 + iota
            hb = h * HD
            att0 = plsc.load_gather(bias2, [boff + rows * H + h])
            att1 = jnp.zeros((16,), jnp.float32)
            for d in range(0, HD, 2):
                c0 = ((d + iota) & (HD - 1)) + hb
                c1 = ((d + 1 + iota) & (HD - 1)) + hb
                att0 = att0 + (plsc.load_gather(qb, [rows, c0])
                               * plsc.load_gather(kb, [rows, c0]))
                att1 = att1 + (plsc.load_gather(qb, [rows, c1])
                               * plsc.load_gather(kb, [rows, c1]))
            ex = jnp.exp(att0 + att1)
            plsc.store_scatter(exbuf, [rows, jnp.full((16,), 0, jnp.int32) + h], ex)

        @plsc.parallel_loop(0, G * H, unroll=2)
        def _scale(i):
            g = i >> 3
            h = i & 7
            rows = g * 16 + iota
            hb = h * HD
            ex = plsc.load_gather(exbuf, [rows, jnp.full((16,), 0, jnp.int32) + h])
            for d in range(HD):
                col = ((d + iota) & (HD - 1)) + hb
                vc = plsc.load_gather(vb, [rows, col])
                plsc.store_scatter(vb, [rows, col], vc * ex)

    def _scat(r, vb, exbuf, sa, sd):
        pltpu.async_copy(vb, acc_sh.at[src2.at[r]], sa, add=True)
        pltpu.async_copy(exbuf, den_sh.at[src2.at[r]], sd, add=True)

    def _wait_scat(vb, exbuf, sa, sd):
        pltpu.make_async_copy(vb, acc_sh.at[src2.at[0]], sa).wait()
        pltpu.make_async_copy(exbuf, den_sh.at[src2.at[0]], sd).wait()

    def _sup(k, _):
        off = base_chunk + k * SUP
        pltpu.sync_copy(src2_hbm.at[pl.ds(off, SUP)], src2)
        pltpu.sync_copy(dst2_hbm.at[pl.ds(off, SUP)], dst2)
        pltpu.sync_copy(bias_hbm.at[pl.ds(off * (C * H), SUP * C * H)], bias2)
        _issue(0, qA, kA, vA, gq0, gk0, gv0)
        _issue(1, qB, kB, vB, gq1, gk1, gv1)

        def _pair(j, _):
            a = 2 * j
            _wait_g(qA, kA, vA, gq0, gk0, gv0)
            _compute(a, qA, kA, vA, exA)
            _scat(a, vA, exA, sac0, sde0)
            _wait_g(qB, kB, vB, gq1, gk1, gv1)
            _wait_scat(vA, exA, sac0, sde0)

            @pl.when(j < NPAIR - 1)
            def _():
                _issue(a + 2, qA, kA, vA, gq0, gk0, gv0)

            _compute(a + 1, qB, kB, vB, exB)
            _scat(a + 1, vB, exB, sac1, sde1)
            _wait_scat(vB, exB, sac1, sde1)

            @pl.when(j < NPAIR - 1)
            def _():
                _issue(a + 3, qB, kB, vB, gq1, gk1, gv1)

            return 0

        lax.fori_loop(0, NPAIR, _pair, 0)
        return 0

    lax.fori_loop(0, NSUP, _sup, 0)

    @pl.when(tile < 16)
    def _():
        off = base_chunk + BCH
        pltpu.sync_copy(src2_hbm.at[pl.ds(off, 1)], src2.at[pl.ds(0, 1)])
        pltpu.sync_copy(dst2_hbm.at[pl.ds(off, 1)], dst2.at[pl.ds(0, 1)])
        pltpu.sync_copy(bias_hbm.at[pl.ds(off * (C * H), C * H)],
                        bias2.at[pl.ds(0, C * H)])
        _issue(0, qA, kA, vA, gq0, gk0, gv0)
        _wait_g(qA, kA, vA, gq0, gk0, gv0)
        _compute(0, qA, kA, vA, exA)
        _scat(0, vA, exA, sac0, sde0)
        _wait_scat(vA, exA, sac0, sde0)

    plsc.subcore_barrier()

    pltpu.sync_copy(acc_sh.at[pl.ds(row0, RS)], acc_out.at[c, pl.ds(row0, RS)])
    pltpu.sync_copy(den_sh.at[pl.ds(row0, RS)], den_out.at[c, pl.ds(row0, RS)])


def _sc_pass(qh, kh, vh, bias1d, src, dst):
    mesh = plsc.VectorSubcoreMesh(core_axis_name="c", subcore_axis_name="s")
    f = pl.kernel(
        _sc_body,
        out_type=(jax.ShapeDtypeStruct((NC, NP_, DF), jnp.float32),
                  jax.ShapeDtypeStruct((NC, NP_, 2 * H), jnp.float32)),
        mesh=mesh,
        compiler_params=pltpu.CompilerParams(needs_layout_passes=False,
                                             use_tc_tiling_on_sc=False),
        scratch_types=[
            pltpu.VMEM((SUP, C), jnp.int32),
            pltpu.VMEM((SUP, C), jnp.int32),
            pltpu.VMEM((SUP * C * H,), jnp.float32),
            pltpu.VMEM((C, DF), jnp.float32),
            pltpu.VMEM((C, DF), jnp.float32),
            pltpu.VMEM((C, DF), jnp.float32),
            pltpu.VMEM((C, DF), jnp.float32),
            pltpu.VMEM((C, DF), jnp.float32),
            pltpu.VMEM((C, DF), jnp.float32),
            pltpu.VMEM((C, 2 * H), jnp.float32),
            pltpu.VMEM((C, 2 * H), jnp.float32),
            pltpu.VMEM_SHARED((NP_, DF), jnp.float32),
            pltpu.VMEM_SHARED((NP_, 2 * H), jnp.float32),
        ] + [pltpu.SemaphoreType.DMA] * 10,
    )
    return f(qh, kh, vh, bias1d,
             src.reshape(E // C, C), dst.reshape(E // C, C))


def _final_body(acc_ref, den_ref, r_ref, wo_ref, bo_ref, o_ref):
    a = acc_ref[0] + acc_ref[1]
    dn = den_ref[0] + den_ref[1]
    dr = jnp.dot(dn, r_ref[...], preferred_element_type=jnp.float32)
    dr = jnp.where(dr == 0.0, 1.0, dr)
    o = a / dr
    o_ref[...] = jnp.dot(o, wo_ref[...],
                         preferred_element_type=jnp.float32) + bo_ref[...]


def _finalize(acc, den, Wo, bo):
    BN = 2000
    grid = (N // BN,)
    rep = jnp.asarray(
        np.vstack([np.kron(np.eye(H), np.ones((1, HD))),
                   np.zeros((H, DF))]), dtype=jnp.float32)
    return pl.pallas_call(
        _final_body,
        grid=grid,
        in_specs=[pl.BlockSpec((NC, BN, DF), lambda i: (0, i, 0)),
                  pl.BlockSpec((NC, BN, 2 * H), lambda i: (0, i, 0)),
                  pl.BlockSpec((2 * H, DF), lambda i: (0, 0)),
                  pl.BlockSpec((DF, DF), lambda i: (0, 0)),
                  pl.BlockSpec((1, DF), lambda i: (0, 0))],
        out_specs=pl.BlockSpec((BN, DF), lambda i: (i, 0)),
        out_shape=jax.ShapeDtypeStruct((N, DF), jnp.float32),
    )(acc, den, rep, Wo, bo.reshape(1, DF))


def kernel(q, k, v, edges, edge_index, Wq, Wk, Wv, Wo, bo, Wb, bb):
    src = edge_index[:, 0]
    dst = edge_index[:, 1]
    qh, kh, vh = _proj(q, k, v, Wq, Wk, Wv)
    bias = _bias(edges, Wb, bb)
    acc, den = _sc_pass(qh, kh, vh, bias.reshape(E * H), src, dst)
    return _finalize(acc, den, Wo, bo)
